# baseline (speedup 1.0000x reference)
import jax
import jax.numpy as jnp
from jax import lax
from jax.experimental import pallas as pl
from jax.experimental.pallas import tpu as pltpu
from jax.experimental.pallas import tpu_sc as plsc

_B, _T, _C = 4, 4096, 1024
_TOP_K = 8
_C_BLK = 256

_SORT8 = (
    (0, 1), (2, 3), (4, 5), (6, 7),
    (0, 2), (1, 3), (4, 6), (5, 7),
    (1, 2), (5, 6),
    (0, 4), (1, 5), (2, 6), (3, 7),
    (2, 4), (3, 5),
    (1, 2), (3, 4), (5, 6),
)

_BITONIC8 = (
    (0, 4), (1, 5), (2, 6), (3, 7),
    (0, 2), (1, 3), (4, 6), (5, 7),
    (0, 1), (2, 3), (4, 5), (6, 7),
)


def _compare_exchange(planes, net):
    planes = list(planes)
    for i, j in net:
        hi = jnp.maximum(planes[i], planes[j])
        lo = jnp.minimum(planes[i], planes[j])
        planes[i], planes[j] = hi, lo
    return planes


_CHUNK = 128
_ROWS = _CHUNK // _TOP_K


def _merge_sorted(carry, planes):
    merged = [jnp.maximum(carry[i], planes[7 - i]) for i in range(_TOP_K)]
    return _compare_exchange(merged, _BITONIC8)


_T_BLK = 1024


def _topk_mean_kernel(conv_ref, gate_ref, out_ref):
    for c0 in range(0, conv_ref.shape[2], _C_BLK):
        def load_sorted(base, c0=c0):
            planes = [
                (conv_ref[0, base + _ROWS * j:base + _ROWS * (j + 1),
                          c0:c0 + _C_BLK]
                 * gate_ref[0, base + _ROWS * j:base + _ROWS * (j + 1),
                            c0:c0 + _C_BLK]
                 ).astype(jnp.bfloat16)
                for j in range(_TOP_K)
            ]
            return _compare_exchange(planes, _SORT8)

        planes = load_sorted(0)
        for i in range(1, _T_BLK // _CHUNK):
            planes = _merge_sorted(planes, load_sorted(i * _CHUNK))
        planes = [p.astype(jnp.float32) for p in planes]
        r = _ROWS
        while r > 1:
            h = r // 2
            a = [p[:h, :] for p in planes]
            b = [p[h:, :] for p in planes]
            planes = [jnp.maximum(a[i], b[7 - i]) for i in range(_TOP_K)]
            planes = _compare_exchange(planes, _BITONIC8)
            r = h
        for j in range(_TOP_K):
            out_ref[0, 0, j, c0:c0 + _C_BLK] = planes[j][0, :]


_SC_LANES = 16
_SC_NC, _SC_NS = 2, 16
_SC_NW = _SC_NC * _SC_NS
_C_SC = 256
_C_TC = _C - _C_SC
_SC_CB = _C_SC // 128
_SC_UNITS = _B * _SC_CB
_SC_SHARDS = _SC_NW // _SC_UNITS
_SC_SHARD_T = _T // _SC_SHARDS
_SC_CHUNK = 64
_SC_NCHUNK = _SC_SHARD_T // _SC_CHUNK


def _hy_sc_body(conv_hbm, gate_hbm, out_hbm,
                conv0, gate0, conv1, gate1, carry_buf, out_buf, sem0, sem1):
    wid = lax.axis_index("s") * _SC_NC + lax.axis_index("c")
    u = wid // _SC_SHARDS
    sh = wid % _SC_SHARDS
    b = u // _SC_CB
    cb = u % _SC_CB
    ch0 = pl.multiple_of(_C_TC + cb * 128, 128)
    row0 = pl.multiple_of(sh * _SC_SHARD_T, 8)
    bufs = ((conv0, gate0, sem0), (conv1, gate1, sem1))

    neg = jnp.full((_SC_LANES,), -jnp.inf, jnp.float32)
    for r in range(64):
        carry_buf[r] = neg

    def issue(ci):
        cbuf, gbuf, sem = bufs[ci % 2]
        rows = pl.ds(row0 + ci * _SC_CHUNK, _SC_CHUNK)
        cols = pl.ds(ch0, 128)
        return (pltpu.async_copy(conv_hbm.at[b, rows, cols], cbuf, sem),
                pltpu.async_copy(gate_hbm.at[b, rows, cols], gbuf, sem))

    pend = issue(0)
    for ci in range(_SC_NCHUNK):
        pend[0].wait()
        pend[1].wait()
        if ci + 1 < _SC_NCHUNK:
            pend = issue(ci + 1)
        cbuf, gbuf, _ = bufs[ci % 2]

        def body(i, dummy, cbuf=cbuf, gbuf=gbuf):
            s = i // 8
            rowb = (i % 8) * 8
            lane = pl.multiple_of(s * _SC_LANES, 16)
            carry = [carry_buf[s * _TOP_K + j] for j in range(_TOP_K)]
            vs = [cbuf[rowb + j, pl.ds(lane, _SC_LANES)]
                  * gbuf[rowb + j, pl.ds(lane, _SC_LANES)]
                  for j in range(_TOP_K)]
            vs = _compare_exchange(vs, _SORT8)
            merged = _merge_sorted(carry, vs)
            for j in range(_TOP_K):
                carry_buf[s * _TOP_K + j] = merged[j]
            return dummy

        lax.fori_loop(0, 64, body, jnp.int32(0))

    for s in range(8):
        for j in range(_TOP_K):
            out_buf[j, pl.ds(s * _SC_LANES, _SC_LANES)] = (
                carry_buf[s * _TOP_K + j])
    pltpu.sync_copy(out_buf, out_hbm.at[wid])


def _sc_topk_shards(local_conv, local_gate):
    return pl.kernel(
        _hy_sc_body,
        out_type=jax.ShapeDtypeStruct((_SC_NW, _TOP_K, 128), jnp.float32),
        mesh=plsc.VectorSubcoreMesh(core_axis_name="c", subcore_axis_name="s"),
        scratch_types=[
            pltpu.VMEM((_SC_CHUNK, 128), jnp.float32),
            pltpu.VMEM((_SC_CHUNK, 128), jnp.float32),
            pltpu.VMEM((_SC_CHUNK, 128), jnp.float32),
            pltpu.VMEM((_SC_CHUNK, 128), jnp.float32),
            pltpu.VMEM((64, _SC_LANES), jnp.float32),
            pltpu.VMEM((_TOP_K, 128), jnp.float32),
            pltpu.SemaphoreType.DMA,
            pltpu.SemaphoreType.DMA,
        ],
    )(local_conv, local_gate)


def _dense_kernel(shards_ref, w_ref, b_ref, out_ref):
    nshard = _T // _T_BLK
    shard_planes = [
        [shards_ref[:, s, j, :] for j in range(_TOP_K)]
        for s in range(nshard)
    ]
    while len(shard_planes) > 1:
        shard_planes = [
            _merge_sorted(shard_planes[2 * i], shard_planes[2 * i + 1])
            for i in range(len(shard_planes) // 2)
        ]
    planes = shard_planes[0]
    acc = planes[0]
    for p in planes[1:]:
        acc = acc + p
    pooled = acc * (1.0 / _TOP_K)
    out = jnp.dot(pooled, w_ref[...], preferred_element_type=jnp.float32)
    out_ref[...] = jnp.maximum(out + b_ref[...], 0.0)


def kernel(local_conv, local_gate, W, b):
    nshard = _T // _T_BLK
    shards_sc = _sc_topk_shards(local_conv, local_gate)
    shards_tc = pl.pallas_call(
        _topk_mean_kernel,
        grid=(_B, nshard),
        in_specs=[
            pl.BlockSpec((1, _T_BLK, _C_TC), lambda i, k: (i, k, 0)),
            pl.BlockSpec((1, _T_BLK, _C_TC), lambda i, k: (i, k, 0)),
        ],
        out_specs=pl.BlockSpec(
            (1, 1, _TOP_K, _C_TC), lambda i, k: (i, k, 0, 0)),
        out_shape=jax.ShapeDtypeStruct(
            (_B, nshard, _TOP_K, _C_TC), jnp.float32),
    )(local_conv, local_gate)

    shards_sc = shards_sc.reshape(_B, _SC_CB, _SC_SHARDS, _TOP_K, 128)
    shards_sc = shards_sc.transpose(0, 2, 3, 1, 4).reshape(
        _B, _SC_SHARDS, _TOP_K, _C_SC)
    shards = jnp.concatenate([shards_tc, shards_sc], axis=-1)

    out = pl.pallas_call(
        _dense_kernel,
        in_specs=[
            pl.BlockSpec((_B, nshard, _TOP_K, _C), lambda: (0, 0, 0, 0)),
            pl.BlockSpec((_C, _C), lambda: (0, 0)),
            pl.BlockSpec((_C,), lambda: (0,)),
        ],
        out_specs=pl.BlockSpec((_B, _C), lambda: (0, 0)),
        out_shape=jax.ShapeDtypeStruct((_B, _C), jnp.float32),
    )(shards, W, b)
    return out


# speedup vs baseline: 1.4452x; 1.4452x over previous
import jax
import jax.numpy as jnp
from jax.experimental import pallas as pl

_B, _T, _C = 4, 4096, 1024
_TOP_K = 8
_C_BLK = 256

_SORT8 = (
    (0, 1), (2, 3), (4, 5), (6, 7),
    (0, 2), (1, 3), (4, 6), (5, 7),
    (1, 2), (5, 6),
    (0, 4), (1, 5), (2, 6), (3, 7),
    (2, 4), (3, 5),
    (1, 2), (3, 4), (5, 6),
)

_BITONIC8 = (
    (0, 4), (1, 5), (2, 6), (3, 7),
    (0, 2), (1, 3), (4, 6), (5, 7),
    (0, 1), (2, 3), (4, 5), (6, 7),
)


def _compare_exchange(planes, net):
    planes = list(planes)
    for i, j in net:
        hi = jnp.maximum(planes[i], planes[j])
        lo = jnp.minimum(planes[i], planes[j])
        planes[i], planes[j] = hi, lo
    return planes


_CHUNK = 128
_ROWS = _CHUNK // _TOP_K


def _merge_sorted(carry, planes):
    merged = [jnp.maximum(carry[i], planes[7 - i]) for i in range(_TOP_K)]
    return _compare_exchange(merged, _BITONIC8)


_T_BLK = 1024


def _topk_mean_kernel(conv_ref, gate_ref, out_ref):
    for c0 in range(0, _C, _C_BLK):
        def load_sorted(base, c0=c0):
            planes = [
                (conv_ref[0, base + _ROWS * j:base + _ROWS * (j + 1),
                          c0:c0 + _C_BLK]
                 * gate_ref[0, base + _ROWS * j:base + _ROWS * (j + 1),
                            c0:c0 + _C_BLK]
                 ).astype(jnp.bfloat16)
                for j in range(_TOP_K)
            ]
            return _compare_exchange(planes, _SORT8)

        planes = load_sorted(0)
        for i in range(1, _T_BLK // _CHUNK):
            planes = _merge_sorted(planes, load_sorted(i * _CHUNK))
        planes = [p.astype(jnp.float32) for p in planes]
        r = _ROWS
        while r > 1:
            h = r // 2
            a = [p[:h, :] for p in planes]
            b = [p[h:, :] for p in planes]
            planes = [jnp.maximum(a[i], b[7 - i]) for i in range(_TOP_K)]
            planes = _compare_exchange(planes, _BITONIC8)
            r = h
        for j in range(_TOP_K):
            out_ref[0, 0, j, c0:c0 + _C_BLK] = planes[j][0, :]


def _dense_kernel(shards_ref, w_ref, b_ref, out_ref):
    nshard = _T // _T_BLK
    shard_planes = [
        [shards_ref[:, s, j, :] for j in range(_TOP_K)]
        for s in range(nshard)
    ]
    while len(shard_planes) > 1:
        shard_planes = [
            _merge_sorted(shard_planes[2 * i], shard_planes[2 * i + 1])
            for i in range(len(shard_planes) // 2)
        ]
    planes = shard_planes[0]
    acc = planes[0]
    for p in planes[1:]:
        acc = acc + p
    pooled = acc * (1.0 / _TOP_K)
    out = jnp.dot(pooled, w_ref[...], preferred_element_type=jnp.float32)
    out_ref[...] = jnp.maximum(out + b_ref[...], 0.0)


def kernel(local_conv, local_gate, W, b):
    nshard = _T // _T_BLK
    shards = pl.pallas_call(
        _topk_mean_kernel,
        grid=(_B, nshard),
        in_specs=[
            pl.BlockSpec((1, _T_BLK, _C), lambda i, k: (i, k, 0)),
            pl.BlockSpec((1, _T_BLK, _C), lambda i, k: (i, k, 0)),
        ],
        out_specs=pl.BlockSpec((1, 1, _TOP_K, _C), lambda i, k: (i, k, 0, 0)),
        out_shape=jax.ShapeDtypeStruct((_B, nshard, _TOP_K, _C), jnp.float32),
    )(local_conv, local_gate)

    out = pl.pallas_call(
        _dense_kernel,
        in_specs=[
            pl.BlockSpec((_B, nshard, _TOP_K, _C), lambda: (0, 0, 0, 0)),
            pl.BlockSpec((_C, _C), lambda: (0, 0)),
            pl.BlockSpec((_C,), lambda: (0,)),
        ],
        out_specs=pl.BlockSpec((_B, _C), lambda: (0, 0)),
        out_shape=jax.ShapeDtypeStruct((_B, _C), jnp.float32),
    )(shards, W, b)
    return out
